# baseline (speedup 1.0000x reference)
import functools

import jax
import jax.numpy as jnp
import numpy as np
from jax import lax
from jax.experimental import pallas as pl
from jax.experimental.pallas import tpu as pltpu
from jax.experimental.pallas import tpu_sc as plsc

B = 16384
V = 100000
D = 128

_NC = 2
_NS = 16
_NW = _NC * _NS
_BPW = B // _NW
_R = 128
_NCH = _BPW // _R
_TP = 137
_L = 16

_MAGIC = np.int32(0x5F3759DF)


def _rsqrt16(x):
    i = plsc.bitcast(x, jnp.int32)
    i = _MAGIC - lax.shift_right_logical(i, 1)
    y = plsc.bitcast(i, jnp.float32)
    hx = x * 0.5
    for _ in range(3):
        y = y * (1.5 - hx * y * y)
    return y


def _sc_fused(idx3, table, cx, cy, ng, w0, w1, gb, nb):
    mesh = plsc.VectorSubcoreMesh(core_axis_name="c", subcore_axis_name="s")

    @functools.partial(
        pl.kernel,
        out_type=jax.ShapeDtypeStruct((D, B), jnp.float32),
        mesh=mesh,
        scratch_types=[
            pltpu.VMEM((_NCH, _R), jnp.int32),
            pltpu.VMEM((2, _R, D), jnp.float32),
            pltpu.VMEM((2, D, _TP), jnp.float32),
            pltpu.VMEM((_BPW,), jnp.float32),
            pltpu.VMEM((_BPW,), jnp.float32),
            pltpu.VMEM((_BPW,), jnp.float32),
            pltpu.VMEM((D,), jnp.float32),
            pltpu.VMEM((D,), jnp.float32),
            pltpu.VMEM((D,), jnp.float32),
            pltpu.VMEM((D,), jnp.float32),
            pltpu.SemaphoreType.DMA,
            pltpu.SemaphoreType.DMA,
        ],
        compiler_params=pltpu.CompilerParams(needs_layout_passes=False),
    )
    def k(idx_hbm, table_hbm, cx_hbm, cy_hbm, ng_hbm, w0_hbm, w1_hbm,
          gb_hbm, nb_hbm, out_hbm,
          idx_v, rows_v, trans_v, cx_v, cy_v, ng_v, w0_v, w1_v, gb_v, nb_v,
          gsem, osem):
        wid = lax.axis_index("s") * _NC + lax.axis_index("c")
        base = wid * _BPW

        pltpu.sync_copy(idx_hbm.at[wid], idx_v)
        pltpu.sync_copy(cx_hbm.at[pl.ds(base, _BPW)], cx_v)
        pltpu.sync_copy(cy_hbm.at[pl.ds(base, _BPW)], cy_v)
        pltpu.sync_copy(ng_hbm.at[pl.ds(base, _BPW)], ng_v)
        pltpu.sync_copy(w0_hbm, w0_v)
        pltpu.sync_copy(w1_hbm, w1_v)
        pltpu.sync_copy(gb_hbm, gb_v)
        pltpu.sync_copy(nb_hbm, nb_v)

        lane = lax.iota(jnp.int32, _L)
        dlanes = [lane + dj * _L for dj in range(D // _L)]
        zeros16 = jnp.zeros((_L,), jnp.float32)

        first = pltpu.async_copy(table_hbm.at[idx_v.at[0]], rows_v.at[0],
                                 gsem)
        pending = [first]
        out_pending = [None, None]

        for c in range(_NCH):
            cb = c % 2
            if c + 1 < _NCH:
                pending.append(
                    pltpu.async_copy(table_hbm.at[idx_v.at[c + 1]],
                                     rows_v.at[(c + 1) % 2], gsem))
            pending.pop(0).wait()
            if out_pending[cb] is not None:
                out_pending[cb].wait()
                out_pending[cb] = None

            def t_body(r, _):
                cbv = jnp.full((_L,), cb, jnp.int32)
                col = jnp.full((_L,), r, jnp.int32)
                for dj in range(D // _L):
                    v = rows_v[cb, r, pl.ds(dj * _L, _L)]
                    plsc.store_scatter(trans_v, [cbv, dlanes[dj], col], v)
                return 0

            lax.fori_loop(0, _R, t_body, 0, unroll=2)

            def g_body(g, _):
                gof = g * _L

                def a_body(d, n2):
                    t = trans_v[cb, d, pl.ds(gof, _L)]
                    return n2 + t * t

                n2 = lax.fori_loop(0, D, a_body, zeros16, unroll=8)
                rn = _rsqrt16(n2)
                ngv = ng_v[pl.ds(c * _R + gof, _L)]
                sc = 1.0 - ngv
                a = cx_v[pl.ds(c * _R + gof, _L)] * sc
                b = cy_v[pl.ds(c * _R + gof, _L)] * sc

                def b_body(dj, _):
                    dof = dj * _L
                    w0b = w0_v[pl.ds(dof, _L)]
                    w1b = w1_v[pl.ds(dof, _L)]
                    gbb = gb_v[pl.ds(dof, _L)]
                    nbb = nb_v[pl.ds(dof, _L)]
                    for l in range(_L):
                        t = trans_v[cb, dof + l, pl.ds(gof, _L)]
                        res = (t * rn + a * w0b[l] + b * w1b[l]
                               + (gbb[l] + ngv * (nbb[l] - gbb[l])))
                        trans_v[cb, dof + l, pl.ds(gof, _L)] = res
                    return 0

                lax.fori_loop(0, D // _L, b_body, 0)
                return 0

            lax.fori_loop(0, _R // _L, g_body, 0)

            out_pending[cb] = pltpu.async_copy(
                trans_v.at[cb, :, pl.ds(0, _R)],
                out_hbm.at[:, pl.ds(base + c * _R, _R)],
                osem)

        for cp in out_pending:
            if cp is not None:
                cp.wait()

    return k(idx3, table, cx, cy, ng, w0, w1, gb, nb)


def kernel(nodes, coords, nogeo, table, geo_W, geo_B, nogeo_embed):
    idx3 = nodes.astype(jnp.int32).reshape(_NW, _NCH, _R)
    cxy = coords.T
    return _sc_fused(idx3, table, cxy[0], cxy[1], nogeo,
                     geo_W[0], geo_W[1], geo_B[0], nogeo_embed[0])
